# Initial kernel scaffold; baseline (speedup 1.0000x reference)
import jax
import jax.numpy as jnp
from jax import lax
from jax.experimental import pallas as pl
from jax.experimental.pallas import tpu as pltpu
from jax.experimental.pallas import tpu_sc as plsc

B, C, H, W = 8, 18, 384, 384
PATCH = 15
PAD = PATCH // 2
NPAIR = B * C
BLK = 16
WB = W // BLK
NW = 32
PPW = (NPAIR + NW - 1) // NW
NQ = 3 * 16 * 2
DENOM = float(NPAIR * PATCH * PATCH * 3)


def _kps_body(bp1_ref, bp2_ref, r1_ref, c1_ref, r2_ref, c2_ref, v_ref):
    iota_r = lax.broadcasted_iota(jnp.int32, (H, W), 0)
    iota_c = lax.broadcasted_iota(jnp.int32, (1, W), 1)

    def amax(x):
        m = jnp.max(x)
        ridx = jnp.min(jnp.where(x == m, iota_r, jnp.int32(H)), axis=0,
                       keepdims=True)
        flat = jnp.min(ridx * W + iota_c)
        return m, flat

    m1, f1 = amax(bp1_ref[0, 0])
    m2, f2 = amax(bp2_ref[0, 0])
    v = jnp.where((m1 > 0.5) & (m2 > 0.5), jnp.float32(1.0), jnp.float32(0.0))
    r1_ref[...] = jnp.full((1, 1, 128), f1 // W, jnp.int32)
    c1_ref[...] = jnp.full((1, 1, 128), f1 % W, jnp.int32)
    r2_ref[...] = jnp.full((1, 1, 128), f2 // W, jnp.int32)
    c2_ref[...] = jnp.full((1, 1, 128), f2 % W, jnp.int32)
    v_ref[...] = jnp.full((1, 1, 128), v, jnp.float32)


def _kps(bp_in, bp_out, interpret=False):
    outs = pl.pallas_call(
        _kps_body,
        grid=(NPAIR,),
        in_specs=[pl.BlockSpec((1, 1, H, W), lambda i: (i // C, i % C, 0, 0)),
                  pl.BlockSpec((1, 1, H, W), lambda i: (i // C, i % C, 0, 0))],
        out_specs=[pl.BlockSpec((1, 1, 128), lambda i: (i, 0, 0))] * 5,
        out_shape=[jax.ShapeDtypeStruct((NPAIR, 1, 128), jnp.int32)] * 4
        + [jax.ShapeDtypeStruct((NPAIR, 1, 128), jnp.float32)],
        interpret=interpret,
    )(bp_in, bp_out)
    return [o[:, 0, 0] for o in outs]


def _sc_body(r1h, c1h, r2h, c2h, vh, tin, tout, outh,
             r1v, c1v, r2v, c2v, vv, idxi, idxo, bufi, bufo, accv,
             semi, semo):
    cid = lax.axis_index("c")
    sid = lax.axis_index("s")
    wid = sid * 2 + cid

    pltpu.sync_copy(r1h, r1v)
    pltpu.sync_copy(c1h, c1v)
    pltpu.sync_copy(r2h, r2v)
    pltpu.sync_copy(c2h, c2v)
    pltpu.sync_copy(vh, vv)

    lanes = lax.iota(jnp.int32, 16)
    dmask = lanes < PATCH
    acc = jnp.zeros((16,), jnp.float32)

    for i in range(PPW):
        p_raw = wid * PPW + i
        ok = (p_raw < NPAIR).astype(jnp.float32)
        p = jnp.minimum(p_raw, NPAIR - 1)
        pv = jnp.full((16,), p, jnp.int32)
        kr1 = plsc.load_gather(r1v, [pv])
        kc1 = plsc.load_gather(c1v, [pv])
        kr2 = plsc.load_gather(r2v, [pv])
        kc2 = plsc.load_gather(c2v, [pv])
        vp = plsc.load_gather(vv, [pv]) * ok
        b3 = (p // C) * 3

        blkA1 = jnp.clip(kc1 - PAD, 0, W - 1) // BLK
        blkB1 = jnp.clip(kc1 + PAD, 0, W - 1) // BLK
        blkA2 = jnp.clip(kc2 - PAD, 0, W - 1) // BLK
        blkB2 = jnp.clip(kc2 + PAD, 0, W - 1) // BLK

        for kr, blkA, blkB, idxref in ((kr1, blkA1, blkB1, idxi),
                                       (kr2, blkA2, blkB2, idxo)):
            for j in range(NQ // 16):
                q = j * 16 + lanes
                ch = q // 32
                rem = q - ch * 32
                dy = rem // 2
                s = rem - dy * 2
                row = jnp.clip(kr - PAD + dy, 0, H - 1)
                blk = jnp.where(s == 0, blkA, blkB)
                idxref[pl.ds(j * 16, 16)] = ((b3 + ch) * H + row) * WB + blk

        cin = pltpu.async_copy(tin.at[idxi], bufi, semi)
        cout = pltpu.async_copy(tout.at[idxo], bufo, semo)
        cin.wait()
        cout.wait()

        def body(t, a):
            ch = t // PATCH
            dy = t - ch * PATCH

            def side_val(kr, kc, blkA, buf):
                r = kr - PAD + dy
                c = kc - PAD + lanes
                inb = (r >= 0) & (r < H) & (c >= 0) & (c < W)
                cc = jnp.clip(c, 0, W - 1)
                hi = cc // BLK - blkA
                rq = ch * 32 + dy * 2 + hi
                rc = cc - (cc // BLK) * BLK
                val = plsc.load_gather(buf, [rq, rc])
                return jnp.where(inb, val, jnp.float32(-1.0))

            vi = side_val(kr1, kc1, blkA1, bufi)
            vo = side_val(kr2, kc2, blkA2, bufo)
            d = vo - vi
            return a + jnp.where(dmask, vp * d * d, jnp.float32(0.0))

        acc = lax.fori_loop(0, 3 * PATCH, body, acc)

    accv[...] = acc
    pltpu.sync_copy(accv, outh.at[wid])


def _sc_patch_loss(r1, c1, r2, c2, vf, t_in, t_out):
    mesh = plsc.VectorSubcoreMesh(core_axis_name="c", subcore_axis_name="s")
    call = pl.kernel(
        _sc_body,
        out_type=jax.ShapeDtypeStruct((NW, BLK), jnp.float32),
        mesh=mesh,
        scratch_types=[
            pltpu.VMEM((NPAIR,), jnp.int32),
            pltpu.VMEM((NPAIR,), jnp.int32),
            pltpu.VMEM((NPAIR,), jnp.int32),
            pltpu.VMEM((NPAIR,), jnp.int32),
            pltpu.VMEM((NPAIR,), jnp.float32),
            pltpu.VMEM((NQ,), jnp.int32),
            pltpu.VMEM((NQ,), jnp.int32),
            pltpu.VMEM((NQ, BLK), jnp.float32),
            pltpu.VMEM((NQ, BLK), jnp.float32),
            pltpu.VMEM((BLK,), jnp.float32),
            pltpu.SemaphoreType.DMA,
            pltpu.SemaphoreType.DMA,
        ],
    )
    return call(r1, c1, r2, c2, vf, t_in, t_out)


@jax.jit
def kernel(img_in, bp_in, img_out, bp_out):
    r1, c1, r2, c2, vf = _kps(bp_in, bp_out)
    t_in = img_in.reshape(B * 3 * H * WB, BLK)
    t_out = img_out.reshape(B * 3 * H * WB, BLK)
    part = _sc_patch_loss(r1, c1, r2, c2, vf, t_in, t_out)
    return jnp.sum(part) * jnp.float32(1.0 / DENOM)


# speedup vs baseline: 2.4207x; 2.4207x over previous
import jax
import jax.numpy as jnp
from jax import lax
from jax.experimental import pallas as pl
from jax.experimental.pallas import tpu as pltpu
from jax.experimental.pallas import tpu_sc as plsc

B, C, H, W = 8, 18, 384, 384
PATCH = 15
PAD = PATCH // 2
NPAIR = B * C
NW = 32
PPW = (NPAIR + NW - 1) // NW
TH, TW = 8, 128
RB = H // TH
CB = W // TW
NTILE = B * 3 * RB * CB
NSLOT = 24
DENOM = float(NPAIR * PATCH * PATCH * 3)


def _kps_body(bp1_ref, bp2_ref, r1_ref, c1_ref, r2_ref, c2_ref, v_ref):
    iota_r = lax.broadcasted_iota(jnp.int32, (H, W), 0)
    iota_c = lax.broadcasted_iota(jnp.int32, (1, W), 1)

    def amax(x):
        m = jnp.max(x)
        ridx = jnp.min(jnp.where(x == m, iota_r, jnp.int32(H)), axis=0,
                       keepdims=True)
        flat = jnp.min(ridx * W + iota_c)
        return m, flat

    m1, f1 = amax(bp1_ref[0, 0])
    m2, f2 = amax(bp2_ref[0, 0])
    v = jnp.where((m1 > 0.5) & (m2 > 0.5), jnp.float32(1.0), jnp.float32(0.0))
    r1_ref[...] = jnp.full((1, 1, 128), f1 // W, jnp.int32)
    c1_ref[...] = jnp.full((1, 1, 128), f1 % W, jnp.int32)
    r2_ref[...] = jnp.full((1, 1, 128), f2 // W, jnp.int32)
    c2_ref[...] = jnp.full((1, 1, 128), f2 % W, jnp.int32)
    v_ref[...] = jnp.full((1, 1, 128), v, jnp.float32)


def _kps(bp_in, bp_out, interpret=False):
    outs = pl.pallas_call(
        _kps_body,
        grid=(NPAIR,),
        in_specs=[pl.BlockSpec((1, 1, H, W), lambda i: (i // C, i % C, 0, 0)),
                  pl.BlockSpec((1, 1, H, W), lambda i: (i // C, i % C, 0, 0))],
        out_specs=[pl.BlockSpec((1, 1, 128), lambda i: (i, 0, 0))] * 5,
        out_shape=[jax.ShapeDtypeStruct((NPAIR, 1, 128), jnp.int32)] * 4
        + [jax.ShapeDtypeStruct((NPAIR, 1, 128), jnp.float32)],
        interpret=interpret,
    )(bp_in, bp_out)
    return [o[:, 0, 0] for o in outs]


def _sc_body(r1h, c1h, r2h, c2h, vh, tin, tout, outh,
             r1v, c1v, r2v, c2v, vv, idxi, idxo, bufi, bufo, accv,
             semi, semo):
    cid = lax.axis_index("c")
    sid = lax.axis_index("s")
    wid = sid * 2 + cid

    pltpu.sync_copy(r1h, r1v)
    pltpu.sync_copy(c1h, c1v)
    pltpu.sync_copy(r2h, r2v)
    pltpu.sync_copy(c2h, c2v)
    pltpu.sync_copy(vh, vv)

    lanes = lax.iota(jnp.int32, 16)
    dmask = lanes < PATCH
    acc = jnp.zeros((16,), jnp.float32)

    for i in range(PPW):
        p_raw = wid * PPW + i
        ok = (p_raw < NPAIR).astype(jnp.float32)
        p = jnp.minimum(p_raw, NPAIR - 1)
        pr = jnp.full((16,), p // 16, jnp.int32)
        pc = jnp.full((16,), p % 16, jnp.int32)
        kr1 = plsc.load_gather(r1v, [pr, pc])
        kc1 = plsc.load_gather(c1v, [pr, pc])
        kr2 = plsc.load_gather(r2v, [pr, pc])
        kc2 = plsc.load_gather(c2v, [pr, pc])
        vp = plsc.load_gather(vv, [pr, pc]) * ok
        b3 = (p // C) * 3

        rb01 = jnp.clip(kr1 - PAD, 0, H - 1) // TH
        cb01 = jnp.clip(kc1 - PAD, 0, W - 1) // TW
        rb02 = jnp.clip(kr2 - PAD, 0, H - 1) // TH
        cb02 = jnp.clip(kc2 - PAD, 0, W - 1) // TW

        for rb0, cb0, idxref in ((rb01, cb01, idxi), (rb02, cb02, idxo)):
            for j in (0, 8):
                s = j + lanes
                ch = s // 8
                u = s - ch * 8
                rb_off = u // 2
                cb_off = u - rb_off * 2
                rb = jnp.clip(rb0 + rb_off, 0, RB - 1)
                cb = jnp.clip(cb0 + cb_off, 0, CB - 1)
                idxref[pl.ds(j, 16)] = ((b3 + ch) * RB + rb) * CB + cb

        cin = pltpu.async_copy(tin.at[idxi], bufi, semi)
        cout = pltpu.async_copy(tout.at[idxo], bufo, semo)
        cin.wait()
        cout.wait()

        def body(t, a):
            ch = t // PATCH
            dy = t - ch * PATCH

            def side_val(kr, kc, rb0, cb0, buf):
                r = kr - PAD + dy
                c = kc - PAD + lanes
                inb = (r >= 0) & (r < H) & (c >= 0) & (c < W)
                rcl = jnp.clip(r, 0, H - 1)
                ccl = jnp.clip(c, 0, W - 1)
                rb = rcl // TH
                cb = ccl // TW
                slot = ch * 8 + (rb - rb0) * 2 + (cb - cb0)
                val = plsc.load_gather(buf, [slot, rcl - rb * TH, ccl - cb * TW])
                return jnp.where(inb, val, jnp.float32(-1.0))

            vi = side_val(kr1, kc1, rb01, cb01, bufi)
            vo = side_val(kr2, kc2, rb02, cb02, bufo)
            d = vo - vi
            return a + jnp.where(dmask, vp * d * d, jnp.float32(0.0))

        acc = lax.fori_loop(0, 3 * PATCH, body, acc)

    accv[...] = acc
    pltpu.sync_copy(accv, outh.at[wid])


def _sc_patch_loss(r1, c1, r2, c2, vf, t_in, t_out):
    mesh = plsc.VectorSubcoreMesh(core_axis_name="c", subcore_axis_name="s")
    call = pl.kernel(
        _sc_body,
        out_type=jax.ShapeDtypeStruct((NW, 16), jnp.float32),
        mesh=mesh,
        compiler_params=pltpu.CompilerParams(needs_layout_passes=False),
        scratch_types=[
            pltpu.VMEM((NPAIR // 16, 16), jnp.int32),
            pltpu.VMEM((NPAIR // 16, 16), jnp.int32),
            pltpu.VMEM((NPAIR // 16, 16), jnp.int32),
            pltpu.VMEM((NPAIR // 16, 16), jnp.int32),
            pltpu.VMEM((NPAIR // 16, 16), jnp.float32),
            pltpu.VMEM((NSLOT,), jnp.int32),
            pltpu.VMEM((NSLOT,), jnp.int32),
            pltpu.VMEM((NSLOT, TH, TW), jnp.float32),
            pltpu.VMEM((NSLOT, TH, TW), jnp.float32),
            pltpu.VMEM((16,), jnp.float32),
            pltpu.SemaphoreType.DMA,
            pltpu.SemaphoreType.DMA,
        ],
    )
    return call(r1.reshape(NPAIR // 16, 16), c1.reshape(NPAIR // 16, 16),
                r2.reshape(NPAIR // 16, 16), c2.reshape(NPAIR // 16, 16),
                vf.reshape(NPAIR // 16, 16), t_in, t_out)


def _tiles(img):
    t = img.reshape(B * 3, RB, TH, CB, TW).transpose(0, 1, 3, 2, 4)
    return t.reshape(NTILE, TH, TW)


@jax.jit
def kernel(img_in, bp_in, img_out, bp_out):
    r1, c1, r2, c2, vf = _kps(bp_in, bp_out)
    part = _sc_patch_loss(r1, c1, r2, c2, vf, _tiles(img_in), _tiles(img_out))
    return jnp.sum(part) * jnp.float32(1.0 / DENOM)


# speedup vs baseline: 2.9407x; 1.2148x over previous
import jax
import jax.numpy as jnp
from jax import lax
from jax.experimental import pallas as pl
from jax.experimental.pallas import tpu as pltpu
from jax.experimental.pallas import tpu_sc as plsc

B, C, H, W = 8, 18, 384, 384
PATCH = 15
PAD = PATCH // 2
NPAIR = B * C
NW = 32
PPW = (NPAIR + NW - 1) // NW
TH, TW = 8, 128
RB = H // TH
CB = W // TW
NTILE = B * 3 * RB * CB
NSLOT = 24
DENOM = float(NPAIR * PATCH * PATCH * 3)


def _kps_body(bp1_ref, bp2_ref, out_ref):
    n_acc = 4

    def argmax2d(ref):
        bv = [None] * n_acc
        bk = [None] * n_acc
        for k in range(RB * CB):
            rb, cb = divmod(k, CB)
            x = ref[0, 0, pl.ds(rb * TH, TH), pl.ds(cb * TW, TW)]
            a = k % n_acc
            if bv[a] is None:
                bv[a] = x
                bk[a] = jnp.full((TH, TW), k, jnp.int32)
            else:
                cond = x > bv[a]
                bv[a] = jnp.where(cond, x, bv[a])
                bk[a] = jnp.where(cond, jnp.int32(k), bk[a])
        mv, mk = bv[0], bk[0]
        for a in range(1, n_acc):
            better = (bv[a] > mv) | ((bv[a] == mv) & (bk[a] < mk))
            mv = jnp.where(better, bv[a], mv)
            mk = jnp.where(better, bk[a], mk)
        s_io = lax.broadcasted_iota(jnp.int32, (TH, TW), 0)
        l_io = lax.broadcasted_iota(jnp.int32, (TH, TW), 1)
        flat = (mk // CB) * (TH * W) + (mk % CB) * TW + s_io * W + l_io
        m = jnp.max(mv)
        fbest = jnp.min(jnp.where(mv == m, flat, jnp.int32(H * W)))
        return m, fbest

    m1, f1 = argmax2d(bp1_ref)
    m2, f2 = argmax2d(bp2_ref)
    v = jnp.where((m1 > 0.5) & (m2 > 0.5), jnp.int32(1), jnp.int32(0))
    lane = lax.broadcasted_iota(jnp.int32, (1, 1, 128), 2)
    out_ref[...] = jnp.where(
        lane == 0, f1 // W,
        jnp.where(lane == 1, f1 % W,
                  jnp.where(lane == 2, f2 // W,
                            jnp.where(lane == 3, f2 % W, v))))


def _kps(bp_in, bp_out, interpret=False):
    return pl.pallas_call(
        _kps_body,
        grid=(NPAIR,),
        in_specs=[pl.BlockSpec((1, 1, H, W), lambda i: (i // C, i % C, 0, 0)),
                  pl.BlockSpec((1, 1, H, W), lambda i: (i // C, i % C, 0, 0))],
        out_specs=pl.BlockSpec((1, 1, 128), lambda i: (i, 0, 0)),
        out_shape=jax.ShapeDtypeStruct((NPAIR, 1, 128), jnp.int32),
        interpret=interpret,
    )(bp_in, bp_out)


def _sc_body(kph, tin, tout, outh,
             kpv, idxi, idxo, bufi, bufo, accv, semi, semo):
    cid = lax.axis_index("c")
    sid = lax.axis_index("s")
    wid = sid * 2 + cid

    lanes = lax.iota(jnp.int32, 16)
    dmask = lanes < PATCH
    zero = jnp.zeros((16,), jnp.int32)
    acc = jnp.zeros((16,), jnp.float32)

    for i in range(PPW):
        p_raw = wid * PPW + i
        ok = (p_raw < NPAIR).astype(jnp.float32)
        p = jnp.minimum(p_raw, NPAIR - 1)
        pltpu.sync_copy(kph.at[p], kpv)
        kr1 = plsc.load_gather(kpv, [zero, zero])
        kc1 = plsc.load_gather(kpv, [zero, zero + 1])
        kr2 = plsc.load_gather(kpv, [zero, zero + 2])
        kc2 = plsc.load_gather(kpv, [zero, zero + 3])
        vp = plsc.load_gather(kpv, [zero, zero + 4]).astype(jnp.float32) * ok
        b3 = (p // C) * 3

        rb01 = jnp.clip(kr1 - PAD, 0, H - 1) // TH
        cb01 = jnp.clip(kc1 - PAD, 0, W - 1) // TW
        rb02 = jnp.clip(kr2 - PAD, 0, H - 1) // TH
        cb02 = jnp.clip(kc2 - PAD, 0, W - 1) // TW

        for rb0, cb0, idxref in ((rb01, cb01, idxi), (rb02, cb02, idxo)):
            for j in (0, 8):
                s = j + lanes
                ch = s // 8
                u = s - ch * 8
                rb_off = u // 2
                cb_off = u - rb_off * 2
                rb = jnp.clip(rb0 + rb_off, 0, RB - 1)
                cb = jnp.clip(cb0 + cb_off, 0, CB - 1)
                idxref[pl.ds(j, 16)] = ((b3 + ch) * RB + rb) * CB + cb

        cin = pltpu.async_copy(tin.at[idxi], bufi, semi)
        cout = pltpu.async_copy(tout.at[idxo], bufo, semo)
        cin.wait()
        cout.wait()

        def body(t, a):
            ch = t // PATCH
            dy = t - ch * PATCH

            def side_val(kr, kc, rb0, cb0, buf):
                r = kr - PAD + dy
                c = kc - PAD + lanes
                inb = (r >= 0) & (r < H) & (c >= 0) & (c < W)
                rcl = jnp.clip(r, 0, H - 1)
                ccl = jnp.clip(c, 0, W - 1)
                rb = rcl // TH
                cb = ccl // TW
                slot = ch * 8 + (rb - rb0) * 2 + (cb - cb0)
                val = plsc.load_gather(buf, [slot, rcl - rb * TH, ccl - cb * TW])
                return jnp.where(inb, val, jnp.float32(-1.0))

            vi = side_val(kr1, kc1, rb01, cb01, bufi)
            vo = side_val(kr2, kc2, rb02, cb02, bufo)
            d = vo - vi
            return a + jnp.where(dmask, vp * d * d, jnp.float32(0.0))

        acc = lax.fori_loop(0, 3 * PATCH, body, acc)

    accv[...] = acc
    pltpu.sync_copy(accv, outh.at[wid])


def _sc_patch_loss(kp, t_in, t_out):
    mesh = plsc.VectorSubcoreMesh(core_axis_name="c", subcore_axis_name="s")
    call = pl.kernel(
        _sc_body,
        out_type=jax.ShapeDtypeStruct((NW, 16), jnp.float32),
        mesh=mesh,
        compiler_params=pltpu.CompilerParams(needs_layout_passes=False),
        scratch_types=[
            pltpu.VMEM((1, 128), jnp.int32),
            pltpu.VMEM((NSLOT,), jnp.int32),
            pltpu.VMEM((NSLOT,), jnp.int32),
            pltpu.VMEM((NSLOT, TH, TW), jnp.float32),
            pltpu.VMEM((NSLOT, TH, TW), jnp.float32),
            pltpu.VMEM((16,), jnp.float32),
            pltpu.SemaphoreType.DMA,
            pltpu.SemaphoreType.DMA,
        ],
    )
    return call(kp, t_in, t_out)


def _tiles(img):
    t = img.reshape(B * 3, RB, TH, CB, TW).transpose(0, 1, 3, 2, 4)
    return t.reshape(NTILE, TH, TW)


@jax.jit
def kernel(img_in, bp_in, img_out, bp_out):
    kp = _kps(bp_in, bp_out)
    part = _sc_patch_loss(kp, _tiles(img_in), _tiles(img_out))
    return jnp.sum(part) * jnp.float32(1.0 / DENOM)


# speedup vs baseline: 2.9423x; 1.0005x over previous
import jax
import jax.numpy as jnp
from jax import lax
from jax.experimental import pallas as pl
from jax.experimental.pallas import tpu as pltpu
from jax.experimental.pallas import tpu_sc as plsc

B, C, H, W = 8, 18, 384, 384
PATCH = 15
PAD = PATCH // 2
NPAIR = B * C
NW = 32
PPW = (NPAIR + NW - 1) // NW
TH, TW = 8, 128
RB = H // TH
CB = W // TW
NTILE = B * 3 * RB * CB
NSLOT = 24
DENOM = float(NPAIR * PATCH * PATCH * 3)


def _kps_body(bp1_ref, bp2_ref, out_ref):
    n_acc = 4

    def argmax2d(ref):
        bv = [None] * n_acc
        bk = [None] * n_acc
        for k in range(RB * CB):
            rb, cb = divmod(k, CB)
            x = ref[0, 0, pl.ds(rb * TH, TH), pl.ds(cb * TW, TW)]
            a = k % n_acc
            if bv[a] is None:
                bv[a] = x
                bk[a] = jnp.full((TH, TW), k, jnp.int32)
            else:
                cond = x > bv[a]
                bv[a] = jnp.where(cond, x, bv[a])
                bk[a] = jnp.where(cond, jnp.int32(k), bk[a])
        mv, mk = bv[0], bk[0]
        for a in range(1, n_acc):
            better = (bv[a] > mv) | ((bv[a] == mv) & (bk[a] < mk))
            mv = jnp.where(better, bv[a], mv)
            mk = jnp.where(better, bk[a], mk)
        s_io = lax.broadcasted_iota(jnp.int32, (TH, TW), 0)
        l_io = lax.broadcasted_iota(jnp.int32, (TH, TW), 1)
        flat = (mk // CB) * (TH * W) + (mk % CB) * TW + s_io * W + l_io
        m = jnp.max(mv)
        fbest = jnp.min(jnp.where(mv == m, flat, jnp.int32(H * W)))
        return m, fbest

    m1, f1 = argmax2d(bp1_ref)
    m2, f2 = argmax2d(bp2_ref)
    v = jnp.where((m1 > 0.5) & (m2 > 0.5), jnp.int32(1), jnp.int32(0))
    lane = lax.broadcasted_iota(jnp.int32, (1, 1, 128), 2)
    out_ref[...] = jnp.where(
        lane == 0, f1 // W,
        jnp.where(lane == 1, f1 % W,
                  jnp.where(lane == 2, f2 // W,
                            jnp.where(lane == 3, f2 % W, v))))


def _kps(bp_in, bp_out, interpret=False):
    return pl.pallas_call(
        _kps_body,
        grid=(NPAIR,),
        in_specs=[pl.BlockSpec((1, 1, H, W), lambda i: (i // C, i % C, 0, 0)),
                  pl.BlockSpec((1, 1, H, W), lambda i: (i // C, i % C, 0, 0))],
        out_specs=pl.BlockSpec((1, 1, 128), lambda i: (i, 0, 0)),
        out_shape=jax.ShapeDtypeStruct((NPAIR, 1, 128), jnp.int32),
        interpret=interpret,
    )(bp_in, bp_out)


def _sc_body(kph, tin, tout, outh,
             kpv, idxi, idxo, bufi, bufo, accv, semi, semo):
    cid = lax.axis_index("c")
    sid = lax.axis_index("s")
    wid = sid * 2 + cid

    pltpu.sync_copy(kph, kpv)

    lanes = lax.iota(jnp.int32, 16)
    dmask = lanes < PATCH
    zero = jnp.zeros((16,), jnp.int32)
    acc = jnp.zeros((16,), jnp.float32)

    for i in range(PPW):
        p_raw = wid * PPW + i
        ok = (p_raw < NPAIR).astype(jnp.float32)
        p = jnp.minimum(p_raw, NPAIR - 1)
        pv = jnp.full((16,), p, jnp.int32)
        kr1 = plsc.load_gather(kpv, [pv, zero])
        kc1 = plsc.load_gather(kpv, [pv, zero + 1])
        kr2 = plsc.load_gather(kpv, [pv, zero + 2])
        kc2 = plsc.load_gather(kpv, [pv, zero + 3])
        vp = plsc.load_gather(kpv, [pv, zero + 4]).astype(jnp.float32) * ok
        b3 = (p // C) * 3

        rb01 = jnp.clip(kr1 - PAD, 0, H - 1) // TH
        cb01 = jnp.clip(kc1 - PAD, 0, W - 1) // TW
        rb02 = jnp.clip(kr2 - PAD, 0, H - 1) // TH
        cb02 = jnp.clip(kc2 - PAD, 0, W - 1) // TW

        for rb0, cb0, idxref in ((rb01, cb01, idxi), (rb02, cb02, idxo)):
            for j in (0, 8):
                s = j + lanes
                ch = s // 8
                u = s - ch * 8
                rb_off = u // 2
                cb_off = u - rb_off * 2
                rb = jnp.clip(rb0 + rb_off, 0, RB - 1)
                cb = jnp.clip(cb0 + cb_off, 0, CB - 1)
                idxref[pl.ds(j, 16)] = ((b3 + ch) * RB + rb) * CB + cb

        cin = pltpu.async_copy(tin.at[idxi], bufi, semi)
        cout = pltpu.async_copy(tout.at[idxo], bufo, semo)
        cin.wait()
        cout.wait()

        def body(t, a):
            ch = t // PATCH
            dy = t - ch * PATCH

            def side_val(kr, kc, rb0, cb0, buf):
                r = kr - PAD + dy
                c = kc - PAD + lanes
                inb = (r >= 0) & (r < H) & (c >= 0) & (c < W)
                rcl = jnp.clip(r, 0, H - 1)
                ccl = jnp.clip(c, 0, W - 1)
                rb = rcl // TH
                cb = ccl // TW
                slot = ch * 8 + (rb - rb0) * 2 + (cb - cb0)
                val = plsc.load_gather(buf, [slot, rcl - rb * TH, ccl - cb * TW])
                return jnp.where(inb, val, jnp.float32(-1.0))

            vi = side_val(kr1, kc1, rb01, cb01, bufi)
            vo = side_val(kr2, kc2, rb02, cb02, bufo)
            d = vo - vi
            return a + jnp.where(dmask, vp * d * d, jnp.float32(0.0))

        acc = lax.fori_loop(0, 3 * PATCH, body, acc)

    accv[...] = acc
    pltpu.sync_copy(accv, outh.at[wid])


def _sc_patch_loss(kp, t_in, t_out):
    mesh = plsc.VectorSubcoreMesh(core_axis_name="c", subcore_axis_name="s")
    call = pl.kernel(
        _sc_body,
        out_type=jax.ShapeDtypeStruct((NW, 16), jnp.float32),
        mesh=mesh,
        compiler_params=pltpu.CompilerParams(needs_layout_passes=False),
        scratch_types=[
            pltpu.VMEM((NPAIR, 128), jnp.int32),
            pltpu.VMEM((NSLOT,), jnp.int32),
            pltpu.VMEM((NSLOT,), jnp.int32),
            pltpu.VMEM((NSLOT, TH, TW), jnp.float32),
            pltpu.VMEM((NSLOT, TH, TW), jnp.float32),
            pltpu.VMEM((16,), jnp.float32),
            pltpu.SemaphoreType.DMA,
            pltpu.SemaphoreType.DMA,
        ],
    )
    return call(kp, t_in, t_out)


def _tiles(img):
    t = img.reshape(B * 3, RB, TH, CB, TW).transpose(0, 1, 3, 2, 4)
    return t.reshape(NTILE, TH, TW)


@jax.jit
def kernel(img_in, bp_in, img_out, bp_out):
    kp = _kps(bp_in, bp_out)[:, 0, :]
    part = _sc_patch_loss(kp, _tiles(img_in), _tiles(img_out))
    return jnp.sum(part) * jnp.float32(1.0 / DENOM)


# speedup vs baseline: 5.3434x; 1.8161x over previous
import jax
import jax.numpy as jnp
from jax import lax
from jax.experimental import pallas as pl
from jax.experimental.pallas import tpu as pltpu
from jax.experimental.pallas import tpu_sc as plsc

B, C, H, W = 8, 18, 384, 384
PATCH = 15
PAD = PATCH // 2
NPAIR = B * C
NW = 32
PPW = (NPAIR + NW - 1) // NW
TH, TW = 8, 128
RB = H // TH
CB = W // TW
NTILE = B * 3 * RB * CB
NROW = NTILE * TH
NSLOT = 96
DENOM = float(NPAIR * PATCH * PATCH * 3)
CPB = 18


def _kps_body(bp1_ref, bp2_ref, out_ref):
    n_acc = 8

    def argmax2d(ref, ch2):
        bv = [None] * n_acc
        bk = [None] * n_acc
        for k in range(RB * CB):
            rb, cb = divmod(k, CB)
            x = ref[0, ch2, pl.ds(rb * TH, TH), pl.ds(cb * TW, TW)]
            a = k % n_acc
            if bv[a] is None:
                bv[a] = x
                bk[a] = jnp.full((TH, TW), k, jnp.int32)
            else:
                cond = x > bv[a]
                bv[a] = jnp.where(cond, x, bv[a])
                bk[a] = jnp.where(cond, jnp.int32(k), bk[a])
        mv, mk = bv[0], bk[0]
        for a in range(1, n_acc):
            better = (bv[a] > mv) | ((bv[a] == mv) & (bk[a] < mk))
            mv = jnp.where(better, bv[a], mv)
            mk = jnp.where(better, bk[a], mk)
        s_io = lax.broadcasted_iota(jnp.int32, (TH, TW), 0)
        l_io = lax.broadcasted_iota(jnp.int32, (TH, TW), 1)
        flat = (mk // CB) * (TH * W) + (mk % CB) * TW + s_io * W + l_io
        m = jnp.max(mv)
        fbest = jnp.min(jnp.where(mv == m, flat, jnp.int32(H * W)))
        return m, fbest

    lane = lax.broadcasted_iota(jnp.int32, (1, 1, 128), 2)
    recs = []
    for ch2 in range(CPB):
        m1, f1 = argmax2d(bp1_ref, ch2)
        m2, f2 = argmax2d(bp2_ref, ch2)
        v = jnp.where((m1 > 0.5) & (m2 > 0.5), jnp.int32(1), jnp.int32(0))
        recs.append(jnp.where(
            lane == 0, f1 // W,
            jnp.where(lane == 1, f1 % W,
                      jnp.where(lane == 2, f2 // W,
                                jnp.where(lane == 3, f2 % W, v)))))
    out_ref[...] = jnp.concatenate(recs, axis=0)


def _kps(bp_in, bp_out, interpret=False):
    cpg = C // CPB
    return pl.pallas_call(
        _kps_body,
        grid=(NPAIR // CPB,),
        in_specs=[pl.BlockSpec((1, CPB, H, W),
                               lambda i: (i // cpg, i % cpg, 0, 0)),
                  pl.BlockSpec((1, CPB, H, W),
                               lambda i: (i // cpg, i % cpg, 0, 0))],
        out_specs=pl.BlockSpec((CPB, 1, 128), lambda i: (i, 0, 0)),
        out_shape=jax.ShapeDtypeStruct((NPAIR, 1, 128), jnp.int32),
        interpret=interpret,
    )(bp_in, bp_out)


def _sc_body(kph, tin, tout, outh,
             kpv, idxi0, idxi1, idxo0, idxo1, bufi0, bufi1, bufo0, bufo1,
             accv, semi0, semi1, semo0, semo1):
    cid = lax.axis_index("c")
    sid = lax.axis_index("s")
    wid = sid * 2 + cid

    pltpu.sync_copy(kph, kpv)

    lanes = lax.iota(jnp.int32, 16)
    dmask = lanes < PATCH
    zero = jnp.zeros((16,), jnp.int32)
    acc = jnp.zeros((16,), jnp.float32)

    idxis = (idxi0, idxi1)
    idxos = (idxo0, idxo1)
    bufis = (bufi0, bufi1)
    bufos = (bufo0, bufo1)
    semis = (semi0, semi1)
    semos = (semo0, semo1)

    def stage(i, slot):
        p_raw = wid * PPW + i
        ok = (p_raw < NPAIR).astype(jnp.float32)
        p = jnp.minimum(p_raw, NPAIR - 1)
        pv = jnp.full((16,), p, jnp.int32)
        kr1 = plsc.load_gather(kpv, [pv, zero])
        kc1 = plsc.load_gather(kpv, [pv, zero + 1])
        kr2 = plsc.load_gather(kpv, [pv, zero + 2])
        kc2 = plsc.load_gather(kpv, [pv, zero + 3])
        vp = plsc.load_gather(kpv, [pv, zero + 4]).astype(jnp.float32) * ok
        b3 = (p // C) * 3
        cb01 = jnp.clip(kc1 - PAD, 0, W - 1) // TW
        cb02 = jnp.clip(kc2 - PAD, 0, W - 1) // TW

        for kr, cb0, idxref in ((kr1, cb01, idxis[slot]),
                                (kr2, cb02, idxos[slot])):
            for j in range(0, NSLOT, 16):
                sq = j + lanes
                ch = sq // 32
                u = sq - ch * 32
                dy = u // 2
                cb_off = u - dy * 2
                r = jnp.clip(kr - PAD + dy, 0, H - 1)
                rb = r // TH
                cb = jnp.clip(cb0 + cb_off, 0, CB - 1)
                idxref[pl.ds(j, 16)] = (((b3 + ch) * RB + rb) * CB + cb) * TH + (r - rb * TH)

        cin = pltpu.async_copy(tin.at[idxis[slot]], bufis[slot], semis[slot])
        cout = pltpu.async_copy(tout.at[idxos[slot]], bufos[slot], semos[slot])
        return (kr1, kc1, kr2, kc2, vp, cb01, cb02, cin, cout)

    meta = stage(0, 0)
    for i in range(PPW):
        slot = i % 2
        nxt = stage(i + 1, 1 - slot) if i + 1 < PPW else None
        kr1, kc1, kr2, kc2, vp, cb01, cb02, cin, cout = meta
        cin.wait()
        cout.wait()
        bufi = bufis[slot]
        bufo = bufos[slot]

        def body(t, a):
            ch = t // PATCH
            dy = t - ch * PATCH

            def side_val(kr, kc, cb0, buf):
                r = kr - PAD + dy
                c = kc - PAD + lanes
                inb = (r >= 0) & (r < H) & (c >= 0) & (c < W)
                ccl = jnp.clip(c, 0, W - 1)
                cb = ccl // TW
                slot_v = ch * 32 + dy * 2 + (cb - cb0)
                val = plsc.load_gather(buf, [slot_v, ccl - cb * TW])
                return jnp.where(inb, val, jnp.float32(-1.0))

            vi = side_val(kr1, kc1, cb01, bufi)
            vo = side_val(kr2, kc2, cb02, bufo)
            d = vo - vi
            return a + jnp.where(dmask, vp * d * d, jnp.float32(0.0))

        acc = lax.fori_loop(0, 3 * PATCH, body, acc)
        meta = nxt

    accv[...] = acc
    pltpu.sync_copy(accv, outh.at[wid])


def _sc_patch_loss(kp, t_in, t_out):
    mesh = plsc.VectorSubcoreMesh(core_axis_name="c", subcore_axis_name="s")
    call = pl.kernel(
        _sc_body,
        out_type=jax.ShapeDtypeStruct((NW, 16), jnp.float32),
        mesh=mesh,
        compiler_params=pltpu.CompilerParams(needs_layout_passes=False),
        scratch_types=[
            pltpu.VMEM((NPAIR, 128), jnp.int32),
            pltpu.VMEM((NSLOT,), jnp.int32),
            pltpu.VMEM((NSLOT,), jnp.int32),
            pltpu.VMEM((NSLOT,), jnp.int32),
            pltpu.VMEM((NSLOT,), jnp.int32),
            pltpu.VMEM((NSLOT, TW), jnp.float32),
            pltpu.VMEM((NSLOT, TW), jnp.float32),
            pltpu.VMEM((NSLOT, TW), jnp.float32),
            pltpu.VMEM((NSLOT, TW), jnp.float32),
            pltpu.VMEM((16,), jnp.float32),
            pltpu.SemaphoreType.DMA,
            pltpu.SemaphoreType.DMA,
            pltpu.SemaphoreType.DMA,
            pltpu.SemaphoreType.DMA,
        ],
    )
    return call(kp, t_in, t_out)


def _tiles(img):
    t = img.reshape(B * 3, RB, TH, CB, TW).transpose(0, 1, 3, 2, 4)
    return t.reshape(NROW, TW)


@jax.jit
def kernel(img_in, bp_in, img_out, bp_out):
    kp = _kps(bp_in, bp_out)[:, 0, :]
    part = _sc_patch_loss(kp, _tiles(img_in), _tiles(img_out))
    return jnp.sum(part) * jnp.float32(1.0 / DENOM)
